# Initial kernel scaffold; baseline (speedup 1.0000x reference)
import jax
import jax.numpy as jnp
from jax.experimental import pallas as pl


def kernel(x, edge_index, batch, W1, att_src1, att_dst1, b1, W2, att_src2, att_dst2, b2, fc_W, fc_b):
    raise NotImplementedError("write your pallas kernel here")


# speedup vs baseline: 4.9583x; 4.9583x over previous
import functools

import jax
import jax.numpy as jnp
import numpy as np
from jax.experimental import pallas as pl
from jax.experimental.pallas import tpu as pltpu

N = 10000
G = 64
H = 4
C = 64
OUT = 8
F_IN = 128

NP = 10240
BLK = 1280
NBLK = NP // BLK
E0 = 320000
EP = 331776
EPT = EP // 16
BK = 128
NEB = EPT // BK
EPS = 1e-16


def _blockdiag2(a0, a1):
    z = jnp.zeros((64,), jnp.float32)
    m = jnp.stack([jnp.concatenate([a0, z]), jnp.concatenate([z, a1])], axis=1)
    return jnp.pad(m, ((0, 0), (0, 6)))


def _att_tables(att):
    return jnp.stack([_blockdiag2(att[0], att[1]), _blockdiag2(att[2], att[3])])


def _dense1_body(x_ref, w_ref, as_ref, ad_ref, h_ref, als_ref, ald_ref):
    h = jnp.dot(x_ref[...], w_ref[0], preferred_element_type=jnp.float32)
    h_ref[...] = h
    als_ref[...] = jnp.dot(h, as_ref[0], preferred_element_type=jnp.float32)[None]
    ald_ref[...] = jnp.dot(h, ad_ref[0], preferred_element_type=jnp.float32)[None]


def _dense1(x_pad, W1h, As, Ad):
    return pl.pallas_call(
        _dense1_body,
        grid=(2, NBLK),
        in_specs=[
            pl.BlockSpec((BLK, F_IN), lambda c, i: (i, 0)),
            pl.BlockSpec((1, F_IN, 128), lambda c, i: (c, 0, 0)),
            pl.BlockSpec((1, 128, 8), lambda c, i: (c, 0, 0)),
            pl.BlockSpec((1, 128, 8), lambda c, i: (c, 0, 0)),
        ],
        out_specs=[
            pl.BlockSpec((BLK, 128), lambda c, i: (c * NBLK + i, 0)),
            pl.BlockSpec((1, BLK, 8), lambda c, i: (c, i, 0)),
            pl.BlockSpec((1, BLK, 8), lambda c, i: (c, i, 0)),
        ],
        out_shape=[
            jax.ShapeDtypeStruct((2 * NP, 128), jnp.float32),
            jax.ShapeDtypeStruct((2, NP, 8), jnp.float32),
            jax.ShapeDtypeStruct((2, NP, 8), jnp.float32),
        ],
    )(x_pad, W1h, As, Ad)


def _dense2_body(o0_ref, o1_ref, d0_ref, d1_ref, b1_ref, w2_ref, as_ref,
                 ad_ref, h2_ref, als_ref, ald_ref):
    def sexp(d_ref):
        s = 1.0 / (d_ref[...] + EPS)
        return jnp.concatenate(
            [jnp.broadcast_to(s[:, 0:1], (BLK, 64)),
             jnp.broadcast_to(s[:, 1:2], (BLK, 64))], axis=1)

    lo = o0_ref[...] * sexp(d0_ref)
    hi = o1_ref[...] * sexp(d1_ref)
    h1 = jnp.concatenate([lo, hi], axis=1) + b1_ref[...]
    h1 = jnp.where(h1 > 0, h1, jnp.exp(jnp.minimum(h1, 0.0)) - 1.0)
    h2 = jnp.dot(h1, w2_ref[0], preferred_element_type=jnp.float32)
    h2_ref[...] = h2
    als_ref[...] = jnp.dot(h2, as_ref[0], preferred_element_type=jnp.float32)[None]
    ald_ref[...] = jnp.dot(h2, ad_ref[0], preferred_element_type=jnp.float32)[None]


def _dense2(o0, o1, d0, d1, b1row, W2h, As2, Ad2):
    return pl.pallas_call(
        _dense2_body,
        grid=(2, NBLK),
        in_specs=[
            pl.BlockSpec((BLK, 128), lambda c, i: (i, 0)),
            pl.BlockSpec((BLK, 128), lambda c, i: (i, 0)),
            pl.BlockSpec((BLK, 2), lambda c, i: (i, 0)),
            pl.BlockSpec((BLK, 2), lambda c, i: (i, 0)),
            pl.BlockSpec((1, 256), lambda c, i: (0, 0)),
            pl.BlockSpec((1, 256, 128), lambda c, i: (c, 0, 0)),
            pl.BlockSpec((1, 128, 8), lambda c, i: (c, 0, 0)),
            pl.BlockSpec((1, 128, 8), lambda c, i: (c, 0, 0)),
        ],
        out_specs=[
            pl.BlockSpec((BLK, 128), lambda c, i: (c * NBLK + i, 0)),
            pl.BlockSpec((1, BLK, 8), lambda c, i: (c, i, 0)),
            pl.BlockSpec((1, BLK, 8), lambda c, i: (c, i, 0)),
        ],
        out_shape=[
            jax.ShapeDtypeStruct((2 * NP, 128), jnp.float32),
            jax.ShapeDtypeStruct((2, NP, 8), jnp.float32),
            jax.ShapeDtypeStruct((2, NP, 8), jnp.float32),
        ],
    )(o0, o1, d0, d1, b1row, W2h, As2, Ad2)


def _pool_body(o0_ref, o1_ref, d0_ref, d1_ref, b2_ref, batch_ref, fcw_ref,
               fcb_ref, out_ref, pooled_ref, counts_ref):
    i = pl.program_id(0)

    def heads(o_ref, d_ref):
        s = 1.0 / (d_ref[...] + EPS)
        lo = o_ref[...][:, :64] * jnp.broadcast_to(s[:, 0:1], (BLK, 64))
        hi = o_ref[...][:, 64:] * jnp.broadcast_to(s[:, 1:2], (BLK, 64))
        return lo + hi

    hm = 0.25 * (heads(o0_ref, d0_ref) + heads(o1_ref, d1_ref)) + b2_ref[...]
    ind = (jax.lax.broadcasted_iota(jnp.int32, (G, BLK), 0)
           == batch_ref[...]).astype(jnp.float32)

    @pl.when(i == 0)
    def _():
        pooled_ref[...] = jnp.zeros((G, 64), jnp.float32)
        counts_ref[...] = jnp.zeros((G, 128), jnp.float32)

    pooled_ref[...] += jnp.dot(ind, hm, preferred_element_type=jnp.float32)
    counts_ref[...] += jnp.dot(ind, jnp.ones((BLK, 128), jnp.float32),
                               preferred_element_type=jnp.float32)

    @pl.when(i == NBLK - 1)
    def _():
        pg = pooled_ref[...] / jnp.maximum(counts_ref[...][:, :64], 1.0)
        out_ref[...] = jnp.dot(pg, fcw_ref[...],
                               preferred_element_type=jnp.float32) + fcb_ref[...]


def _pool_fc(o0, o1, d0, d1, b2row, batch2d, fc_W, fc_b):
    return pl.pallas_call(
        _pool_body,
        grid=(NBLK,),
        in_specs=[
            pl.BlockSpec((BLK, 128), lambda i: (i, 0)),
            pl.BlockSpec((BLK, 128), lambda i: (i, 0)),
            pl.BlockSpec((BLK, 2), lambda i: (i, 0)),
            pl.BlockSpec((BLK, 2), lambda i: (i, 0)),
            pl.BlockSpec((1, 64), lambda i: (0, 0)),
            pl.BlockSpec((1, BLK), lambda i: (0, i)),
            pl.BlockSpec((64, 8), lambda i: (0, 0)),
            pl.BlockSpec((1, 8), lambda i: (0, 0)),
        ],
        out_specs=pl.BlockSpec((G, 8), lambda i: (0, 0)),
        out_shape=jax.ShapeDtypeStruct((G, 8), jnp.float32),
        scratch_shapes=[pltpu.VMEM((G, 64), jnp.float32),
                        pltpu.VMEM((G, 128), jnp.float32)],
    )(o0, o1, d0, d1, b2row, batch2d, fc_W, fc_b)


def _edge_phase(h_all, als, ald, m4, srcp, dstp):
    a_s = jnp.concatenate([als[0, :, :2], als[1, :, :2]], axis=1)
    a_d = jnp.concatenate([ald[0, :, :2], ald[1, :, :2]], axis=1)
    e = a_s[srcp] + a_d[dstp]
    e = jnp.maximum(e, 0.2 * e) - m4
    w = jnp.exp(e)
    den = jax.ops.segment_sum(w, dstp, num_segments=NP)
    hfull = jnp.concatenate([h_all[:NP], h_all[NP:]], axis=1)
    msg = hfull[srcp] * jnp.repeat(w, 64, axis=1)
    raw = jax.ops.segment_sum(msg, dstp, num_segments=NP)
    return raw[:, :128], raw[:, 128:], den[:, :2], den[:, 2:]


def _stab(als, ald):
    ms = jnp.max(als[:, :N, :2], axis=1)
    md = jnp.max(ald[:, :N, :2], axis=1)
    s = ms.reshape(4) + md.reshape(4)
    return jnp.maximum(s, 0.2 * s)


def kernel(x, edge_index, batch, W1, att_src1, att_dst1, b1,
           W2, att_src2, att_dst2, b2, fc_W, fc_b):
    x_pad = jnp.pad(x, ((0, NP - N), (0, 0)))
    loop = jnp.arange(N, dtype=jnp.int32)
    fill = jnp.full((EP - E0 - N,), N, jnp.int32)
    srcp = jnp.concatenate([edge_index[0].astype(jnp.int32), loop, fill])
    dstp = jnp.concatenate([edge_index[1].astype(jnp.int32), loop, fill])
    batch2d = jnp.pad(batch.astype(jnp.int32), (0, NP - N),
                      constant_values=127).reshape(1, NP)

    W1h = W1.reshape(1, F_IN, 2, 128).transpose(2, 0, 1, 3).reshape(2, F_IN, 128)
    W2h = W2.reshape(1, 256, 2, 128).transpose(2, 0, 1, 3).reshape(2, 256, 128)

    h_all, als, ald = _dense1(x_pad, W1h, _att_tables(att_src1),
                              _att_tables(att_dst1))
    m4 = _stab(als, ald)
    o0, o1, d0, d1 = _edge_phase(h_all, als, ald, m4, srcp, dstp)

    h2_all, als2, ald2 = _dense2(o0, o1, d0, d1, b1.reshape(1, 256), W2h,
                                 _att_tables(att_src2), _att_tables(att_dst2))
    m4b = _stab(als2, ald2)
    p0, p1, e0, e1 = _edge_phase(h2_all, als2, ald2, m4b, srcp, dstp)

    return _pool_fc(p0, p1, e0, e1, b2.reshape(1, 64), batch2d, fc_W,
                    fc_b.reshape(1, 8))


# speedup vs baseline: 21.3016x; 4.2962x over previous
import functools

import jax
import jax.numpy as jnp
import numpy as np
from jax import lax
from jax.experimental import pallas as pl
from jax.experimental.pallas import tpu as pltpu
from jax.experimental.pallas import tpu_sc as plsc

N = 10000
G = 64
H = 4
C = 64
OUT = 8
F_IN = 128

NP = 10240
BLK = 1280
NBLK = NP // BLK
E0 = 320000
EP = 331776
EPT = EP // 16
BK = 128
NEB = EPT // BK
EPS = 1e-16
NP2 = NP * 2
RPT = NP // 16


def _att_tables(att):
    return jnp.pad(att[:, :, None], ((0, 0), (0, 0), (0, 7)))


def _quarters(W):
    k = W.shape[0]
    return W.reshape(k, 4, 64).transpose(1, 0, 2)


def _dense1_body(x_ref, w_ref, as_ref, ad_ref, h_ref, als_ref, ald_ref):
    hq = jnp.dot(x_ref[...], w_ref[0], preferred_element_type=jnp.float32)
    h_ref[...] = hq
    als_ref[...] = jnp.dot(hq, as_ref[0], preferred_element_type=jnp.float32)[None]
    ald_ref[...] = jnp.dot(hq, ad_ref[0], preferred_element_type=jnp.float32)[None]


def _dense1(x_pad, W1q, As, Ad):
    return pl.pallas_call(
        _dense1_body,
        grid=(4, NBLK),
        in_specs=[
            pl.BlockSpec((BLK, F_IN), lambda h, i: (i, 0)),
            pl.BlockSpec((1, F_IN, 64), lambda h, i: (h, 0, 0)),
            pl.BlockSpec((1, 64, 8), lambda h, i: (h, 0, 0)),
            pl.BlockSpec((1, 64, 8), lambda h, i: (h, 0, 0)),
        ],
        out_specs=[
            pl.BlockSpec((BLK, 64), lambda h, i: (h * NBLK + i, 0)),
            pl.BlockSpec((1, BLK, 8), lambda h, i: (h, i, 0)),
            pl.BlockSpec((1, BLK, 8), lambda h, i: (h, i, 0)),
        ],
        out_shape=[
            jax.ShapeDtypeStruct((4 * NP, 64), jnp.float32),
            jax.ShapeDtypeStruct((4, NP, 8), jnp.float32),
            jax.ShapeDtypeStruct((4, NP, 8), jnp.float32),
        ],
    )(x_pad, W1q, As, Ad)


def _dense2_body(o0_ref, o1_ref, o2_ref, o3_ref, d_ref, b1_ref, w2_ref,
                 as_ref, ad_ref, h2_ref, als_ref, ald_ref):
    s = 1.0 / (d_ref[...] + EPS)
    parts = []
    for h, o_ref in enumerate((o0_ref, o1_ref, o2_ref, o3_ref)):
        parts.append(o_ref[...] * jnp.broadcast_to(s[:, h:h + 1], (BLK, 64)))
    h1 = jnp.concatenate(parts, axis=1) + b1_ref[...]
    h1 = jnp.where(h1 > 0, h1, jnp.exp(jnp.minimum(h1, 0.0)) - 1.0)
    h2q = jnp.dot(h1, w2_ref[0], preferred_element_type=jnp.float32)
    h2_ref[...] = h2q
    als_ref[...] = jnp.dot(h2q, as_ref[0], preferred_element_type=jnp.float32)[None]
    ald_ref[...] = jnp.dot(h2q, ad_ref[0], preferred_element_type=jnp.float32)[None]


def _dense2(oA, oB, d4, b1row, W2q, As2, Ad2):
    return pl.pallas_call(
        _dense2_body,
        grid=(4, NBLK),
        in_specs=[
            pl.BlockSpec((BLK, 64), lambda h, i: (i, 0)),
            pl.BlockSpec((BLK, 64), lambda h, i: (i, 0)),
            pl.BlockSpec((BLK, 64), lambda h, i: (NBLK + i, 0)),
            pl.BlockSpec((BLK, 64), lambda h, i: (NBLK + i, 0)),
            pl.BlockSpec((BLK, 8), lambda h, i: (i, 0)),
            pl.BlockSpec((1, 256), lambda h, i: (0, 0)),
            pl.BlockSpec((1, 256, 64), lambda h, i: (h, 0, 0)),
            pl.BlockSpec((1, 64, 8), lambda h, i: (h, 0, 0)),
            pl.BlockSpec((1, 64, 8), lambda h, i: (h, 0, 0)),
        ],
        out_specs=[
            pl.BlockSpec((BLK, 64), lambda h, i: (h * NBLK + i, 0)),
            pl.BlockSpec((1, BLK, 8), lambda h, i: (h, i, 0)),
            pl.BlockSpec((1, BLK, 8), lambda h, i: (h, i, 0)),
        ],
        out_shape=[
            jax.ShapeDtypeStruct((4 * NP, 64), jnp.float32),
            jax.ShapeDtypeStruct((4, NP, 8), jnp.float32),
            jax.ShapeDtypeStruct((4, NP, 8), jnp.float32),
        ],
    )(oA, oB, oA, oB, d4, b1row, W2q, As2, Ad2)


def _pool_body(o0_ref, o1_ref, o2_ref, o3_ref, d_ref, b2_ref, batch_ref,
               fcw_ref, fcb_ref, out_ref, pooled_ref, counts_ref):
    i = pl.program_id(0)
    s = 1.0 / (d_ref[...] + EPS)
    hm = jnp.zeros((BLK, 64), jnp.float32)
    for h, o_ref in enumerate((o0_ref, o1_ref, o2_ref, o3_ref)):
        hm = hm + o_ref[...] * jnp.broadcast_to(s[:, h:h + 1], (BLK, 64))
    hm = 0.25 * hm + b2_ref[...]
    ind = (jax.lax.broadcasted_iota(jnp.int32, (G, BLK), 0)
           == batch_ref[...]).astype(jnp.float32)

    @pl.when(i == 0)
    def _():
        pooled_ref[...] = jnp.zeros((G, 64), jnp.float32)
        counts_ref[...] = jnp.zeros((G, 128), jnp.float32)

    pooled_ref[...] += jnp.dot(ind, hm, preferred_element_type=jnp.float32)
    counts_ref[...] += jnp.dot(ind, jnp.ones((BLK, 128), jnp.float32),
                               preferred_element_type=jnp.float32)

    @pl.when(i == NBLK - 1)
    def _():
        pg = pooled_ref[...] / jnp.maximum(counts_ref[...][:, :64], 1.0)
        out_ref[...] = jnp.dot(pg, fcw_ref[...],
                               preferred_element_type=jnp.float32) + fcb_ref[...]


def _pool_fc(oA, oB, d4, b2row, batch2d, fc_W, fc_b):
    return pl.pallas_call(
        _pool_body,
        grid=(NBLK,),
        in_specs=[
            pl.BlockSpec((BLK, 64), lambda i: (i, 0)),
            pl.BlockSpec((BLK, 64), lambda i: (i, 0)),
            pl.BlockSpec((BLK, 64), lambda i: (NBLK + i, 0)),
            pl.BlockSpec((BLK, 64), lambda i: (NBLK + i, 0)),
            pl.BlockSpec((BLK, 8), lambda i: (i, 0)),
            pl.BlockSpec((1, 64), lambda i: (0, 0)),
            pl.BlockSpec((1, BLK), lambda i: (0, i)),
            pl.BlockSpec((64, 8), lambda i: (0, 0)),
            pl.BlockSpec((1, 8), lambda i: (0, 0)),
        ],
        out_specs=pl.BlockSpec((G, 8), lambda i: (0, 0)),
        out_shape=jax.ShapeDtypeStruct((G, 8), jnp.float32),
        scratch_shapes=[pltpu.VMEM((G, 64), jnp.float32),
                        pltpu.VMEM((G, 128), jnp.float32)],
    )(oA, oB, oA, oB, d4, b2row, batch2d, fc_W, fc_b.reshape(1, 8))


def _edge_body(j, srcR_hbm, dstR_hbm, h4_hbm, asf_hbm, adf_hbm, m_hbm,
               outr_hbm, den_hbm,
               src_b, dst_b, asf_v, adf_v, m_v, w_v, rows_v, zer_v, zden_v,
               acc_sp, den_sp, sem):
    c = lax.axis_index("c")
    s = lax.axis_index("s")

    pltpu.sync_copy(asf_hbm.at[pl.ds(c * NP, NP)], asf_v)
    pltpu.sync_copy(adf_hbm.at[pl.ds(c * NP, NP)], adf_v)
    pltpu.sync_copy(m_hbm, m_v)

    zv = jnp.zeros((16,), jnp.float32)
    lane = lax.broadcasted_iota(jnp.int32, (16,), 0)

    def zrow(i, carry):
        for k in range(4):
            zer_v[i, pl.ds(k * 16, 16)] = zv
        return carry

    lax.fori_loop(0, 16, zrow, 0)

    def zd(i, carry):
        zden_v[pl.ds(i * 16, 16)] = zv
        return carry

    lax.fori_loop(0, RPT // 16, zd, 0)

    def zacc(k, carry):
        pltpu.sync_copy(zer_v, acc_sp.at[pl.ds(s * RPT + k * 16, 16)])
        return carry

    lax.fori_loop(0, RPT // 16, zacc, 0)
    pltpu.sync_copy(zden_v, den_sp.at[pl.ds(s * RPT, RPT)])
    plsc.subcore_barrier()

    col0 = jnp.zeros((16,), jnp.int32)
    m0 = plsc.load_gather(m_v, [col0 + c])
    tb = (c * 16 + s) * NEB
    tl = s * NEB
    hbase = 2 * c + j

    def blk(b, carry):
        pltpu.sync_copy(srcR_hbm.at[tl + b], src_b)
        pltpu.sync_copy(dstR_hbm.at[tl + b], dst_b)

        for g in range(8):
            sl = pl.ds(g * 16, 16)
            sv = src_b[sl]
            dv = dst_b[sl]
            a = plsc.load_gather(asf_v, [sv])
            d = plsc.load_gather(adf_v, [dv])
            e = a + d
            w = jnp.exp(jnp.maximum(e, 0.2 * e) - m0)
            w_v[sl] = w
            src_b[sl] = sv + hbase * NP

        pltpu.async_copy(h4_hbm.at[src_b], rows_v, sem).wait()

        def edge_j(jj, cc):
            w0 = plsc.load_gather(w_v, [col0 + jj])
            for k in range(4):
                sl = pl.ds(k * 16, 16)
                rows_v[jj, sl] = rows_v[jj, sl] * w0
            return cc

        lax.fori_loop(0, BK, edge_j, 0)

        pltpu.sync_copy(rows_v, acc_sp.at[dst_b], add=True)
        pltpu.sync_copy(w_v, den_sp.at[dst_b], add=True)
        return carry

    lax.fori_loop(0, NEB, blk, 0)
    plsc.subcore_barrier()

    base = c * NP + s * RPT
    pltpu.sync_copy(acc_sp.at[pl.ds(s * RPT, RPT)],
                    outr_hbm.at[pl.ds(base, RPT)])
    pltpu.sync_copy(den_sp.at[pl.ds(s * RPT, RPT)],
                    den_hbm.at[pl.ds(base, RPT)])


_SC_MESH = plsc.VectorSubcoreMesh(core_axis_name="c", subcore_axis_name="s",
                                  num_cores=2, num_subcores=16)


def _edge_sc(j, srcR, dstR, h4, asf, adf, m16):
    f = pl.kernel(
        functools.partial(_edge_body, j),
        out_type=[jax.ShapeDtypeStruct((NP2, 64), jnp.float32),
                  jax.ShapeDtypeStruct((NP2,), jnp.float32)],
        mesh=_SC_MESH,
        compiler_params=pltpu.CompilerParams(needs_layout_passes=False,
                                             use_tc_tiling_on_sc=False),
        scratch_types=[
            pltpu.VMEM((BK,), jnp.int32),
            pltpu.VMEM((BK,), jnp.int32),
            pltpu.VMEM((NP,), jnp.float32),
            pltpu.VMEM((NP,), jnp.float32),
            pltpu.VMEM((16,), jnp.float32),
            pltpu.VMEM((BK,), jnp.float32),
            pltpu.VMEM((BK, 64), jnp.float32),
            pltpu.VMEM((16, 64), jnp.float32),
            pltpu.VMEM((RPT,), jnp.float32),
            pltpu.VMEM_SHARED((NP, 64), jnp.float32),
            pltpu.VMEM_SHARED((NP,), jnp.float32),
            pltpu.SemaphoreType.DMA,
        ],
    )
    return f(srcR, dstR, h4, asf, adf, m16)


def _stab(als, ald):
    ms = jnp.max(als[:, :N, 0], axis=1)
    md = jnp.max(ald[:, :N, 0], axis=1)
    s = ms + md
    return jnp.maximum(s, 0.2 * s)


def _edge_layer(srcR, dstR, h4, als, ald):
    m4 = _stab(als, ald)
    outs, dens = [], []
    for j in (0, 1):
        asf = als[jnp.array([j, j + 2]), :, 0].reshape(NP2)
        adf = ald[jnp.array([j, j + 2]), :, 0].reshape(NP2)
        m16 = jnp.pad(jnp.stack([m4[j], m4[j + 2]]), (0, 14))
        o, d = _edge_sc(j, srcR, dstR, h4, asf, adf, m16)
        outs.append(o)
        dens.append(d)
    d4 = jnp.pad(jnp.stack(
        [dens[0][:NP], dens[1][:NP], dens[0][NP:], dens[1][NP:]], axis=1),
        ((0, 0), (0, 4)), constant_values=1.0)
    return outs[0], outs[1], d4


def kernel(x, edge_index, batch, W1, att_src1, att_dst1, b1,
           W2, att_src2, att_dst2, b2, fc_W, fc_b):
    x_pad = jnp.pad(x, ((0, NP - N), (0, 0)))
    loop = jnp.arange(N, dtype=jnp.int32)
    fill = jnp.full((EP - E0 - N,), N, jnp.int32)
    srcp = jnp.concatenate([edge_index[0].astype(jnp.int32), loop, fill])
    dstp = jnp.concatenate([edge_index[1].astype(jnp.int32), loop, fill])
    srcR = srcp.reshape(16 * NEB, BK)
    dstR = dstp.reshape(16 * NEB, BK)
    batch2d = jnp.pad(batch.astype(jnp.int32), (0, NP - N),
                      constant_values=127).reshape(1, NP)

    h4, als, ald = _dense1(x_pad, _quarters(W1), _att_tables(att_src1),
                           _att_tables(att_dst1))
    oA, oB, d4 = _edge_layer(srcR, dstR, h4, als, ald)

    h4b, als2, ald2 = _dense2(oA, oB, d4, b1.reshape(1, 256), _quarters(W2),
                              _att_tables(att_src2), _att_tables(att_dst2))
    pA, pB, e4 = _edge_layer(srcR, dstR, h4b, als2, ald2)

    return _pool_fc(pA, pB, e4, b2.reshape(1, 64), batch2d, fc_W, fc_b)


# speedup vs baseline: 27.2547x; 1.2795x over previous
import functools

import jax
import jax.numpy as jnp
import numpy as np
from jax import lax
from jax.experimental import pallas as pl
from jax.experimental.pallas import tpu as pltpu
from jax.experimental.pallas import tpu_sc as plsc

N = 10000
G = 64
H = 4
C = 64
OUT = 8
F_IN = 128

NP = 10240
BLK = 1280
NBLK = NP // BLK
E0 = 320000
EP = 331776
EPT = EP // 16
BK = 128
NEB = EPT // BK
EPS = 1e-16
NP2 = NP * 2
RPT = NP // 16


def _att_tables(att):
    return jnp.pad(att[:, :, None], ((0, 0), (0, 0), (0, 7)))


def _quarters(W):
    k = W.shape[0]
    return W.reshape(k, 4, 64).transpose(1, 0, 2)


def _dense1_body(x_ref, w_ref, as_ref, ad_ref, h_ref, als_ref, ald_ref):
    hq = jnp.dot(x_ref[...], w_ref[0], preferred_element_type=jnp.float32)
    h_ref[...] = hq
    als_ref[...] = jnp.dot(hq, as_ref[0], preferred_element_type=jnp.float32)[None]
    ald_ref[...] = jnp.dot(hq, ad_ref[0], preferred_element_type=jnp.float32)[None]


def _dense1(x_pad, W1q, As, Ad):
    return pl.pallas_call(
        _dense1_body,
        grid=(4, NBLK),
        in_specs=[
            pl.BlockSpec((BLK, F_IN), lambda h, i: (i, 0)),
            pl.BlockSpec((1, F_IN, 64), lambda h, i: (h, 0, 0)),
            pl.BlockSpec((1, 64, 8), lambda h, i: (h, 0, 0)),
            pl.BlockSpec((1, 64, 8), lambda h, i: (h, 0, 0)),
        ],
        out_specs=[
            pl.BlockSpec((BLK, 64), lambda h, i: (h * NBLK + i, 0)),
            pl.BlockSpec((1, BLK, 8), lambda h, i: (h, i, 0)),
            pl.BlockSpec((1, BLK, 8), lambda h, i: (h, i, 0)),
        ],
        out_shape=[
            jax.ShapeDtypeStruct((4 * NP, 64), jnp.float32),
            jax.ShapeDtypeStruct((4, NP, 8), jnp.float32),
            jax.ShapeDtypeStruct((4, NP, 8), jnp.float32),
        ],
    )(x_pad, W1q, As, Ad)


def _dense2_body(o0_ref, o1_ref, o2_ref, o3_ref, d_ref, b1_ref, w2_ref,
                 as_ref, ad_ref, h2_ref, als_ref, ald_ref):
    s = 1.0 / (d_ref[...] + EPS)
    parts = []
    for h, o_ref in enumerate((o0_ref, o1_ref, o2_ref, o3_ref)):
        parts.append(o_ref[...] * jnp.broadcast_to(s[:, h:h + 1], (BLK, 64)))
    h1 = jnp.concatenate(parts, axis=1) + b1_ref[...]
    h1 = jnp.where(h1 > 0, h1, jnp.exp(jnp.minimum(h1, 0.0)) - 1.0)
    h2q = jnp.dot(h1, w2_ref[0], preferred_element_type=jnp.float32)
    h2_ref[...] = h2q
    als_ref[...] = jnp.dot(h2q, as_ref[0], preferred_element_type=jnp.float32)[None]
    ald_ref[...] = jnp.dot(h2q, ad_ref[0], preferred_element_type=jnp.float32)[None]


def _dense2(oA, oB, d4, b1row, W2q, As2, Ad2):
    return pl.pallas_call(
        _dense2_body,
        grid=(4, NBLK),
        in_specs=[
            pl.BlockSpec((BLK, 64), lambda h, i: (i, 0)),
            pl.BlockSpec((BLK, 64), lambda h, i: (i, 0)),
            pl.BlockSpec((BLK, 64), lambda h, i: (NBLK + i, 0)),
            pl.BlockSpec((BLK, 64), lambda h, i: (NBLK + i, 0)),
            pl.BlockSpec((BLK, 8), lambda h, i: (i, 0)),
            pl.BlockSpec((1, 256), lambda h, i: (0, 0)),
            pl.BlockSpec((1, 256, 64), lambda h, i: (h, 0, 0)),
            pl.BlockSpec((1, 64, 8), lambda h, i: (h, 0, 0)),
            pl.BlockSpec((1, 64, 8), lambda h, i: (h, 0, 0)),
        ],
        out_specs=[
            pl.BlockSpec((BLK, 64), lambda h, i: (h * NBLK + i, 0)),
            pl.BlockSpec((1, BLK, 8), lambda h, i: (h, i, 0)),
            pl.BlockSpec((1, BLK, 8), lambda h, i: (h, i, 0)),
        ],
        out_shape=[
            jax.ShapeDtypeStruct((4 * NP, 64), jnp.float32),
            jax.ShapeDtypeStruct((4, NP, 8), jnp.float32),
            jax.ShapeDtypeStruct((4, NP, 8), jnp.float32),
        ],
    )(oA, oB, oA, oB, d4, b1row, W2q, As2, Ad2)


def _pool_body(o0_ref, o1_ref, o2_ref, o3_ref, d_ref, b2_ref, batch_ref,
               fcw_ref, fcb_ref, out_ref, pooled_ref, counts_ref):
    i = pl.program_id(0)
    s = 1.0 / (d_ref[...] + EPS)
    hm = jnp.zeros((BLK, 64), jnp.float32)
    for h, o_ref in enumerate((o0_ref, o1_ref, o2_ref, o3_ref)):
        hm = hm + o_ref[...] * jnp.broadcast_to(s[:, h:h + 1], (BLK, 64))
    hm = 0.25 * hm + b2_ref[...]
    ind = (jax.lax.broadcasted_iota(jnp.int32, (G, BLK), 0)
           == batch_ref[...]).astype(jnp.float32)

    @pl.when(i == 0)
    def _():
        pooled_ref[...] = jnp.zeros((G, 64), jnp.float32)
        counts_ref[...] = jnp.zeros((G, 128), jnp.float32)

    pooled_ref[...] += jnp.dot(ind, hm, preferred_element_type=jnp.float32)
    counts_ref[...] += jnp.dot(ind, jnp.ones((BLK, 128), jnp.float32),
                               preferred_element_type=jnp.float32)

    @pl.when(i == NBLK - 1)
    def _():
        pg = pooled_ref[...] / jnp.maximum(counts_ref[...][:, :64], 1.0)
        out_ref[...] = jnp.dot(pg, fcw_ref[...],
                               preferred_element_type=jnp.float32) + fcb_ref[...]


def _pool_fc(oA, oB, d4, b2row, batch2d, fc_W, fc_b):
    return pl.pallas_call(
        _pool_body,
        grid=(NBLK,),
        in_specs=[
            pl.BlockSpec((BLK, 64), lambda i: (i, 0)),
            pl.BlockSpec((BLK, 64), lambda i: (i, 0)),
            pl.BlockSpec((BLK, 64), lambda i: (NBLK + i, 0)),
            pl.BlockSpec((BLK, 64), lambda i: (NBLK + i, 0)),
            pl.BlockSpec((BLK, 8), lambda i: (i, 0)),
            pl.BlockSpec((1, 64), lambda i: (0, 0)),
            pl.BlockSpec((1, BLK), lambda i: (0, i)),
            pl.BlockSpec((64, 8), lambda i: (0, 0)),
            pl.BlockSpec((1, 8), lambda i: (0, 0)),
        ],
        out_specs=pl.BlockSpec((G, 8), lambda i: (0, 0)),
        out_shape=jax.ShapeDtypeStruct((G, 8), jnp.float32),
        scratch_shapes=[pltpu.VMEM((G, 64), jnp.float32),
                        pltpu.VMEM((G, 128), jnp.float32)],
    )(oA, oB, oA, oB, d4, b2row, batch2d, fc_W, fc_b.reshape(1, 8))


def _edge_body(j, srcR_hbm, dstR_hbm, h4_hbm, asf_hbm, adf_hbm, m_hbm,
               outr_hbm, den_hbm,
               src_b, dst_b, asf_v, adf_v, m_v, w_v, rows_v, zer_v, zden_v,
               acc_sp, den_sp, sem):
    c = lax.axis_index("c")
    s = lax.axis_index("s")

    pltpu.sync_copy(asf_hbm.at[pl.ds(c * NP, NP)], asf_v)
    pltpu.sync_copy(adf_hbm.at[pl.ds(c * NP, NP)], adf_v)
    pltpu.sync_copy(m_hbm, m_v)

    zv = jnp.zeros((16,), jnp.float32)
    lane = lax.broadcasted_iota(jnp.int32, (16,), 0)

    def zrow(i, carry):
        for k in range(4):
            zer_v[i, pl.ds(k * 16, 16)] = zv
        return carry

    lax.fori_loop(0, 16, zrow, 0)

    def zd(i, carry):
        zden_v[pl.ds(i * 16, 16)] = zv
        return carry

    lax.fori_loop(0, RPT // 16, zd, 0)

    def zacc(k, carry):
        pltpu.sync_copy(zer_v, acc_sp.at[pl.ds(s * RPT + k * 16, 16)])
        return carry

    lax.fori_loop(0, RPT // 16, zacc, 0)
    pltpu.sync_copy(zden_v, den_sp.at[pl.ds(s * RPT, RPT)])
    plsc.subcore_barrier()

    col0 = jnp.zeros((16,), jnp.int32)
    m0 = plsc.load_gather(m_v, [col0 + c])
    tl = s * NEB
    hbase = 2 * c + j

    def prefetch(q, b):
        pltpu.sync_copy(srcR_hbm.at[tl + b], src_b[q])
        pltpu.sync_copy(dstR_hbm.at[tl + b], dst_b[q])
        for g in range(8):
            sl = pl.ds(g * 16, 16)
            src_b[q][sl] = src_b[q][sl] + hbase * NP
        return pltpu.async_copy(h4_hbm.at[src_b[q]], rows_v[q], sem[q])

    def process(p):
        for g in range(8):
            sl = pl.ds(g * 16, 16)
            sv = src_b[p][sl] - hbase * NP
            dv = dst_b[p][sl]
            a = plsc.load_gather(asf_v, [sv])
            d = plsc.load_gather(adf_v, [dv])
            e = a + d
            w_v[sl] = jnp.exp(jnp.maximum(e, 0.2 * e) - m0)

        def edge_j(jj, cc):
            w0 = plsc.load_gather(w_v, [col0 + jj])
            for k in range(4):
                sl = pl.ds(k * 16, 16)
                rows_v[p][jj, sl] = rows_v[p][jj, sl] * w0
            return cc

        lax.fori_loop(0, BK, edge_j, 0)
        pltpu.sync_copy(rows_v[p], acc_sp.at[dst_b[p]], add=True)
        pltpu.sync_copy(w_v, den_sp.at[dst_b[p]], add=True)

    prefetch(0, 0).wait()

    def blk2(i, carry):
        b = i * 2
        cp1 = prefetch(1, b + 1)
        process(0)
        cp1.wait()
        cp0 = prefetch(0, b + 2)
        process(1)
        cp0.wait()
        return carry

    lax.fori_loop(0, (NEB - 2) // 2, blk2, 0)
    cpl = prefetch(1, NEB - 1)
    process(0)
    cpl.wait()
    process(1)
    plsc.subcore_barrier()

    base = c * NP + s * RPT
    pltpu.sync_copy(acc_sp.at[pl.ds(s * RPT, RPT)],
                    outr_hbm.at[pl.ds(base, RPT)])
    pltpu.sync_copy(den_sp.at[pl.ds(s * RPT, RPT)],
                    den_hbm.at[pl.ds(base, RPT)])


_SC_MESH = plsc.VectorSubcoreMesh(core_axis_name="c", subcore_axis_name="s",
                                  num_cores=2, num_subcores=16)


def _edge_sc(j, srcR, dstR, h4, asf, adf, m16):
    f = pl.kernel(
        functools.partial(_edge_body, j),
        out_type=[jax.ShapeDtypeStruct((NP2, 64), jnp.float32),
                  jax.ShapeDtypeStruct((NP2,), jnp.float32)],
        mesh=_SC_MESH,
        compiler_params=pltpu.CompilerParams(needs_layout_passes=False,
                                             use_tc_tiling_on_sc=False),
        scratch_types=[
            [pltpu.VMEM((BK,), jnp.int32)] * 2,
            [pltpu.VMEM((BK,), jnp.int32)] * 2,
            pltpu.VMEM((NP,), jnp.float32),
            pltpu.VMEM((NP,), jnp.float32),
            pltpu.VMEM((16,), jnp.float32),
            pltpu.VMEM((BK,), jnp.float32),
            [pltpu.VMEM((BK, 64), jnp.float32)] * 2,
            pltpu.VMEM((16, 64), jnp.float32),
            pltpu.VMEM((RPT,), jnp.float32),
            pltpu.VMEM_SHARED((NP, 64), jnp.float32),
            pltpu.VMEM_SHARED((NP,), jnp.float32),
            [pltpu.SemaphoreType.DMA] * 2,
        ],
    )
    return f(srcR, dstR, h4, asf, adf, m16)


def _stab(als, ald):
    ms = jnp.max(als[:, :N, 0], axis=1)
    md = jnp.max(ald[:, :N, 0], axis=1)
    s = ms + md
    return jnp.maximum(s, 0.2 * s)


def _edge_layer(srcR, dstR, h4, als, ald):
    m4 = _stab(als, ald)
    outs, dens = [], []
    for j in (0, 1):
        asf = als[jnp.array([j, j + 2]), :, 0].reshape(NP2)
        adf = ald[jnp.array([j, j + 2]), :, 0].reshape(NP2)
        m16 = jnp.pad(jnp.stack([m4[j], m4[j + 2]]), (0, 14))
        o, d = _edge_sc(j, srcR, dstR, h4, asf, adf, m16)
        outs.append(o)
        dens.append(d)
    d4 = jnp.pad(jnp.stack(
        [dens[0][:NP], dens[1][:NP], dens[0][NP:], dens[1][NP:]], axis=1),
        ((0, 0), (0, 4)), constant_values=1.0)
    return outs[0], outs[1], d4


def kernel(x, edge_index, batch, W1, att_src1, att_dst1, b1,
           W2, att_src2, att_dst2, b2, fc_W, fc_b):
    x_pad = jnp.pad(x, ((0, NP - N), (0, 0)))
    loop = jnp.arange(N, dtype=jnp.int32)
    fill = jnp.full((EP - E0 - N,), N, jnp.int32)
    srcp = jnp.concatenate([edge_index[0].astype(jnp.int32), loop, fill])
    dstp = jnp.concatenate([edge_index[1].astype(jnp.int32), loop, fill])
    srcR = srcp.reshape(16 * NEB, BK)
    dstR = dstp.reshape(16 * NEB, BK)
    batch2d = jnp.pad(batch.astype(jnp.int32), (0, NP - N),
                      constant_values=127).reshape(1, NP)

    h4, als, ald = _dense1(x_pad, _quarters(W1), _att_tables(att_src1),
                           _att_tables(att_dst1))
    oA, oB, d4 = _edge_layer(srcR, dstR, h4, als, ald)

    h4b, als2, ald2 = _dense2(oA, oB, d4, b1.reshape(1, 256), _quarters(W2),
                              _att_tables(att_src2), _att_tables(att_dst2))
    pA, pB, e4 = _edge_layer(srcR, dstR, h4b, als2, ald2)

    return _pool_fc(pA, pB, e4, b2.reshape(1, 64), batch2d, fc_W, fc_b)


# speedup vs baseline: 28.2829x; 1.0377x over previous
import functools

import jax
import jax.numpy as jnp
import numpy as np
from jax import lax
from jax.experimental import pallas as pl
from jax.experimental.pallas import tpu as pltpu
from jax.experimental.pallas import tpu_sc as plsc

N = 10000
G = 64
H = 4
C = 64
OUT = 8
F_IN = 128

NP = 10240
BLK = 1280
NBLK = NP // BLK
E0 = 320000
EP = 331776
EPT = EP // 16
BK = 128
NEB = EPT // BK
EPS = 1e-16
NP2 = NP * 2
RPT = NP // 16


def _att_tables(att):
    return jnp.pad(att[:, :, None], ((0, 0), (0, 0), (0, 7)))


def _quarters(W):
    k = W.shape[0]
    return W.reshape(k, 4, 64).transpose(1, 0, 2)


def _dense1_body(x_ref, w_ref, as_ref, ad_ref, h_ref, als_ref, ald_ref):
    hq = jnp.dot(x_ref[...], w_ref[0], preferred_element_type=jnp.float32)
    h_ref[...] = hq
    als_ref[...] = jnp.dot(hq, as_ref[0], preferred_element_type=jnp.float32)[None]
    ald_ref[...] = jnp.dot(hq, ad_ref[0], preferred_element_type=jnp.float32)[None]


def _dense1(x_pad, W1q, As, Ad):
    return pl.pallas_call(
        _dense1_body,
        grid=(4, NBLK),
        in_specs=[
            pl.BlockSpec((BLK, F_IN), lambda h, i: (i, 0)),
            pl.BlockSpec((1, F_IN, 64), lambda h, i: (h, 0, 0)),
            pl.BlockSpec((1, 64, 8), lambda h, i: (h, 0, 0)),
            pl.BlockSpec((1, 64, 8), lambda h, i: (h, 0, 0)),
        ],
        out_specs=[
            pl.BlockSpec((BLK, 64), lambda h, i: (h * NBLK + i, 0)),
            pl.BlockSpec((1, BLK, 8), lambda h, i: (h, i, 0)),
            pl.BlockSpec((1, BLK, 8), lambda h, i: (h, i, 0)),
        ],
        out_shape=[
            jax.ShapeDtypeStruct((4 * NP, 64), jnp.float32),
            jax.ShapeDtypeStruct((4, NP, 8), jnp.float32),
            jax.ShapeDtypeStruct((4, NP, 8), jnp.float32),
        ],
    )(x_pad, W1q, As, Ad)


def _dense2_body(o0_ref, o1_ref, o2_ref, o3_ref, d_ref, b1_ref, w2_ref,
                 as_ref, ad_ref, h2_ref, als_ref, ald_ref):
    s = 1.0 / (d_ref[...] + EPS)
    parts = []
    for h, o_ref in enumerate((o0_ref, o1_ref, o2_ref, o3_ref)):
        parts.append(o_ref[...] * jnp.broadcast_to(s[:, h:h + 1], (BLK, 64)))
    h1 = jnp.concatenate(parts, axis=1) + b1_ref[...]
    h1 = jnp.where(h1 > 0, h1, jnp.exp(jnp.minimum(h1, 0.0)) - 1.0)
    h2q = jnp.dot(h1, w2_ref[0], preferred_element_type=jnp.float32)
    h2_ref[...] = h2q
    als_ref[...] = jnp.dot(h2q, as_ref[0], preferred_element_type=jnp.float32)[None]
    ald_ref[...] = jnp.dot(h2q, ad_ref[0], preferred_element_type=jnp.float32)[None]


def _dense2(oA, oB, d4, b1row, W2q, As2, Ad2):
    return pl.pallas_call(
        _dense2_body,
        grid=(4, NBLK),
        in_specs=[
            pl.BlockSpec((BLK, 64), lambda h, i: (i, 0)),
            pl.BlockSpec((BLK, 64), lambda h, i: (i, 0)),
            pl.BlockSpec((BLK, 64), lambda h, i: (NBLK + i, 0)),
            pl.BlockSpec((BLK, 64), lambda h, i: (NBLK + i, 0)),
            pl.BlockSpec((BLK, 8), lambda h, i: (i, 0)),
            pl.BlockSpec((1, 256), lambda h, i: (0, 0)),
            pl.BlockSpec((1, 256, 64), lambda h, i: (h, 0, 0)),
            pl.BlockSpec((1, 64, 8), lambda h, i: (h, 0, 0)),
            pl.BlockSpec((1, 64, 8), lambda h, i: (h, 0, 0)),
        ],
        out_specs=[
            pl.BlockSpec((BLK, 64), lambda h, i: (h * NBLK + i, 0)),
            pl.BlockSpec((1, BLK, 8), lambda h, i: (h, i, 0)),
            pl.BlockSpec((1, BLK, 8), lambda h, i: (h, i, 0)),
        ],
        out_shape=[
            jax.ShapeDtypeStruct((4 * NP, 64), jnp.float32),
            jax.ShapeDtypeStruct((4, NP, 8), jnp.float32),
            jax.ShapeDtypeStruct((4, NP, 8), jnp.float32),
        ],
    )(oA, oB, oA, oB, d4, b1row, W2q, As2, Ad2)


def _pool_body(o0_ref, o1_ref, o2_ref, o3_ref, d_ref, b2_ref, batch_ref,
               fcw_ref, fcb_ref, out_ref, pooled_ref, counts_ref):
    i = pl.program_id(0)
    s = 1.0 / (d_ref[...] + EPS)
    hm = jnp.zeros((BLK, 64), jnp.float32)
    for h, o_ref in enumerate((o0_ref, o1_ref, o2_ref, o3_ref)):
        hm = hm + o_ref[...] * jnp.broadcast_to(s[:, h:h + 1], (BLK, 64))
    hm = 0.25 * hm + b2_ref[...]
    ind = (jax.lax.broadcasted_iota(jnp.int32, (G, BLK), 0)
           == batch_ref[...]).astype(jnp.float32)

    @pl.when(i == 0)
    def _():
        pooled_ref[...] = jnp.zeros((G, 64), jnp.float32)
        counts_ref[...] = jnp.zeros((G, 128), jnp.float32)

    pooled_ref[...] += jnp.dot(ind, hm, preferred_element_type=jnp.float32)
    counts_ref[...] += jnp.dot(ind, jnp.ones((BLK, 128), jnp.float32),
                               preferred_element_type=jnp.float32)

    @pl.when(i == NBLK - 1)
    def _():
        pg = pooled_ref[...] / jnp.maximum(counts_ref[...][:, :64], 1.0)
        out_ref[...] = jnp.dot(pg, fcw_ref[...],
                               preferred_element_type=jnp.float32) + fcb_ref[...]


def _pool_fc(oA, oB, d4, b2row, batch2d, fc_W, fc_b):
    return pl.pallas_call(
        _pool_body,
        grid=(NBLK,),
        in_specs=[
            pl.BlockSpec((BLK, 64), lambda i: (i, 0)),
            pl.BlockSpec((BLK, 64), lambda i: (i, 0)),
            pl.BlockSpec((BLK, 64), lambda i: (NBLK + i, 0)),
            pl.BlockSpec((BLK, 64), lambda i: (NBLK + i, 0)),
            pl.BlockSpec((BLK, 8), lambda i: (i, 0)),
            pl.BlockSpec((1, 64), lambda i: (0, 0)),
            pl.BlockSpec((1, BLK), lambda i: (0, i)),
            pl.BlockSpec((64, 8), lambda i: (0, 0)),
            pl.BlockSpec((1, 8), lambda i: (0, 0)),
        ],
        out_specs=pl.BlockSpec((G, 8), lambda i: (0, 0)),
        out_shape=jax.ShapeDtypeStruct((G, 8), jnp.float32),
        scratch_shapes=[pltpu.VMEM((G, 64), jnp.float32),
                        pltpu.VMEM((G, 128), jnp.float32)],
    )(oA, oB, oA, oB, d4, b2row, batch2d, fc_W, fc_b.reshape(1, 8))


def _edge_body(j, srcR_hbm, dstR_hbm, h4_hbm, asf_hbm, adf_hbm, m_hbm,
               outr_hbm, den_hbm,
               src_b, dst_b, asf_v, adf_v, m_v, w_v, rows_v, zer_v, zden_v,
               acc_sp, den_sp, sem):
    c = lax.axis_index("c")
    s = lax.axis_index("s")

    pltpu.sync_copy(asf_hbm.at[pl.ds(c * NP, NP)], asf_v)
    pltpu.sync_copy(adf_hbm.at[pl.ds(c * NP, NP)], adf_v)
    pltpu.sync_copy(m_hbm, m_v)

    zv = jnp.zeros((16,), jnp.float32)
    lane = lax.broadcasted_iota(jnp.int32, (16,), 0)

    def zrow(i, carry):
        for k in range(4):
            zer_v[i, pl.ds(k * 16, 16)] = zv
        return carry

    lax.fori_loop(0, 16, zrow, 0)

    def zd(i, carry):
        zden_v[pl.ds(i * 16, 16)] = zv
        return carry

    lax.fori_loop(0, RPT // 16, zd, 0)

    def zacc(k, carry):
        pltpu.sync_copy(zer_v, acc_sp.at[pl.ds(s * RPT + k * 16, 16)])
        return carry

    lax.fori_loop(0, RPT // 16, zacc, 0)
    pltpu.sync_copy(zden_v, den_sp.at[pl.ds(s * RPT, RPT)])
    plsc.subcore_barrier()

    col0 = jnp.zeros((16,), jnp.int32)
    m0 = plsc.load_gather(m_v, [col0 + c])
    tl = s * NEB
    hbase = 2 * c + j

    def prefetch(q, b):
        pltpu.sync_copy(srcR_hbm.at[tl + b], src_b[q])
        pltpu.sync_copy(dstR_hbm.at[tl + b], dst_b[q])
        for g in range(8):
            sl = pl.ds(g * 16, 16)
            src_b[q][sl] = src_b[q][sl] + hbase * NP
        return pltpu.async_copy(h4_hbm.at[src_b[q]], rows_v[q], sem[q])

    def process(p):
        for g in range(8):
            sl = pl.ds(g * 16, 16)
            sv = src_b[p][sl] - hbase * NP
            dv = dst_b[p][sl]
            a = plsc.load_gather(asf_v, [sv])
            d = plsc.load_gather(adf_v, [dv])
            e = a + d
            w_v[sl] = jnp.exp(jnp.maximum(e, 0.2 * e) - m0)

        def edge_j(j2, cc):
            for u in range(4):
                jj = j2 * 4 + u
                w0 = plsc.load_gather(w_v, [col0 + jj])
                for k in range(4):
                    sl = pl.ds(k * 16, 16)
                    rows_v[p][jj, sl] = rows_v[p][jj, sl] * w0
            return cc

        lax.fori_loop(0, BK // 4, edge_j, 0)
        pltpu.sync_copy(rows_v[p], acc_sp.at[dst_b[p]], add=True)
        pltpu.sync_copy(w_v, den_sp.at[dst_b[p]], add=True)

    prefetch(0, 0).wait()

    def blk2(i, carry):
        b = i * 2
        cp1 = prefetch(1, b + 1)
        process(0)
        cp1.wait()
        cp0 = prefetch(0, b + 2)
        process(1)
        cp0.wait()
        return carry

    lax.fori_loop(0, (NEB - 2) // 2, blk2, 0)
    cpl = prefetch(1, NEB - 1)
    process(0)
    cpl.wait()
    process(1)
    plsc.subcore_barrier()

    base = c * NP + s * RPT
    pltpu.sync_copy(acc_sp.at[pl.ds(s * RPT, RPT)],
                    outr_hbm.at[pl.ds(base, RPT)])
    pltpu.sync_copy(den_sp.at[pl.ds(s * RPT, RPT)],
                    den_hbm.at[pl.ds(base, RPT)])


_SC_MESH = plsc.VectorSubcoreMesh(core_axis_name="c", subcore_axis_name="s",
                                  num_cores=2, num_subcores=16)


def _edge_sc(j, srcR, dstR, h4, asf, adf, m16):
    f = pl.kernel(
        functools.partial(_edge_body, j),
        out_type=[jax.ShapeDtypeStruct((NP2, 64), jnp.float32),
                  jax.ShapeDtypeStruct((NP2,), jnp.float32)],
        mesh=_SC_MESH,
        compiler_params=pltpu.CompilerParams(needs_layout_passes=False,
                                             use_tc_tiling_on_sc=False),
        scratch_types=[
            [pltpu.VMEM((BK,), jnp.int32)] * 2,
            [pltpu.VMEM((BK,), jnp.int32)] * 2,
            pltpu.VMEM((NP,), jnp.float32),
            pltpu.VMEM((NP,), jnp.float32),
            pltpu.VMEM((16,), jnp.float32),
            pltpu.VMEM((BK,), jnp.float32),
            [pltpu.VMEM((BK, 64), jnp.float32)] * 2,
            pltpu.VMEM((16, 64), jnp.float32),
            pltpu.VMEM((RPT,), jnp.float32),
            pltpu.VMEM_SHARED((NP, 64), jnp.float32),
            pltpu.VMEM_SHARED((NP,), jnp.float32),
            [pltpu.SemaphoreType.DMA] * 2,
        ],
    )
    return f(srcR, dstR, h4, asf, adf, m16)


def _stab(als, ald):
    ms = jnp.max(als[:, :N, 0], axis=1)
    md = jnp.max(ald[:, :N, 0], axis=1)
    s = ms + md
    return jnp.maximum(s, 0.2 * s)


def _edge_layer(srcR, dstR, h4, als, ald):
    m4 = _stab(als, ald)
    outs, dens = [], []
    for j in (0, 1):
        asf = als[jnp.array([j, j + 2]), :, 0].reshape(NP2)
        adf = ald[jnp.array([j, j + 2]), :, 0].reshape(NP2)
        m16 = jnp.pad(jnp.stack([m4[j], m4[j + 2]]), (0, 14))
        o, d = _edge_sc(j, srcR, dstR, h4, asf, adf, m16)
        outs.append(o)
        dens.append(d)
    d4 = jnp.pad(jnp.stack(
        [dens[0][:NP], dens[1][:NP], dens[0][NP:], dens[1][NP:]], axis=1),
        ((0, 0), (0, 4)), constant_values=1.0)
    return outs[0], outs[1], d4


def kernel(x, edge_index, batch, W1, att_src1, att_dst1, b1,
           W2, att_src2, att_dst2, b2, fc_W, fc_b):
    x_pad = jnp.pad(x, ((0, NP - N), (0, 0)))
    loop = jnp.arange(N, dtype=jnp.int32)
    fill = jnp.full((EP - E0 - N,), N, jnp.int32)
    srcp = jnp.concatenate([edge_index[0].astype(jnp.int32), loop, fill])
    dstp = jnp.concatenate([edge_index[1].astype(jnp.int32), loop, fill])
    srcR = srcp.reshape(16 * NEB, BK)
    dstR = dstp.reshape(16 * NEB, BK)
    batch2d = jnp.pad(batch.astype(jnp.int32), (0, NP - N),
                      constant_values=127).reshape(1, NP)

    h4, als, ald = _dense1(x_pad, _quarters(W1), _att_tables(att_src1),
                           _att_tables(att_dst1))
    oA, oB, d4 = _edge_layer(srcR, dstR, h4, als, ald)

    h4b, als2, ald2 = _dense2(oA, oB, d4, b1.reshape(1, 256), _quarters(W2),
                              _att_tables(att_src2), _att_tables(att_dst2))
    pA, pB, e4 = _edge_layer(srcR, dstR, h4b, als2, ald2)

    return _pool_fc(pA, pB, e4, b2.reshape(1, 64), batch2d, fc_W, fc_b)
